# baseline (speedup 1.0000x reference)
import functools

import jax
import jax.numpy as jnp
from jax import lax
from jax.experimental import pallas as pl
from jax.experimental.pallas import tpu as pltpu
from jax.experimental.pallas import tpu_sc as plsc

_NC = 2
_NS = 16
_NW = _NC * _NS
_LANES = 16

_C = 8
_PW = _C * 256
_EW = 176
_G = 16


def _ssp_sc_kernel(n_patches: int):
  n_per_w = n_patches // _NW
  n_chunks = n_per_w // _G

  mesh = plsc.VectorSubcoreMesh(core_axis_name="c", subcore_axis_name="s")

  @functools.partial(
      pl.kernel,
      out_type=jax.ShapeDtypeStruct((n_patches * _EW,), jnp.float32),
      mesh=mesh,
      scratch_types=[
          pltpu.VMEM((_G * _PW,), jnp.float32),
          pltpu.VMEM((_G * _EW,), jnp.float32),
      ],
      compiler_params=pltpu.CompilerParams(needs_layout_passes=False),
  )
  def ssp(patch_hbm, emb_hbm, buf, ebuf):
    wid = lax.axis_index("s") * _NC + lax.axis_index("c")

    def make_consts():
      lane = lax.iota(jnp.int32, _LANES)
      base4 = (lane // 4) * 64 + (lane % 4) * 4
      base2 = []
      for v in range(2):
        gid = v * _LANES + lane
        c = gid // 4
        i = (lane // 2) % 2
        j = lane % 2
        per_d = []
        for di in range(2):
          for dj in range(2):
            per_d.append(48 + c * 16 + (2 * i + di) * 4 + (2 * j + dj))
        base2.append(per_d)
      base1 = [16 + jnp.minimum(lane, 7) * 4 + q for q in range(4)]
      return lane, base4, base2, base1

    def chunk_body(ci, carry):
      patch0 = wid * n_per_w + ci * _G
      pltpu.sync_copy(patch_hbm.at[pl.ds(patch0 * _PW, _G * _PW)], buf)

      def p_body(p, c2):
        lane, base4, base2, base1 = make_consts()
        pb = p * _PW
        eb = p * _EW
        for ch in range(_C):
          off = pb + ch * 256
          acc = None
          for dy in range(4):
            for dx in range(4):
              v = plsc.load_gather(buf, [base4 + (off + 16 * dy + dx)])
              acc = v if acc is None else jnp.maximum(acc, v)
          ebuf[pl.ds(eb + 48 + ch * 16, _LANES)] = acc
        for v in range(2):
          acc = None
          for d in range(4):
            g = plsc.load_gather(ebuf, [base2[v][d] + eb])
            acc = g if acc is None else jnp.maximum(acc, g)
          ebuf[pl.ds(eb + 16 + v * _LANES, _LANES)] = acc
        acc = None
        for q in range(4):
          g = plsc.load_gather(ebuf, [base1[q] + eb])
          acc = g if acc is None else jnp.maximum(acc, g)
        ebuf[pl.ds(eb, _LANES)] = jnp.where(lane < 8, acc, 0.0)
        return c2

      lax.fori_loop(0, _G, p_body, 0, unroll=False)
      pltpu.sync_copy(ebuf, emb_hbm.at[pl.ds(patch0 * _EW, _G * _EW)])
      return carry

    lax.fori_loop(0, n_chunks, chunk_body, 0, unroll=False)

  return ssp


def _router_tc_kernel(emb_block: int):
  def body(thr_ref, emb_ref, keys_ref, out_ref):
    x = emb_ref[...]
    s = jnp.sum(x * x, axis=1, keepdims=True)
    x = x / jnp.maximum(jnp.sqrt(s), 1e-12)
    logits = lax.dot_general(
        x, keys_ref[...], (((1,), (1,)), ((), ())),
        preferred_element_type=jnp.float32)
    m = jnp.max(logits, axis=1, keepdims=True)
    e = jnp.exp(logits - m)
    w = e / jnp.sum(e, axis=1, keepdims=True)
    t = thr_ref[0]
    wf = jnp.where(w > t, w, 0.0)
    out_ref[...] = wf / (jnp.sum(wf, axis=1, keepdims=True) + 1e-8)

  return body


def kernel(patch, keys, threshold):
  n = patch.shape[0]
  flat = patch.reshape(n * _PW)

  emb_flat = _ssp_sc_kernel(n)(flat)
  emb = emb_flat.reshape(n, _EW)

  zeros8 = jnp.zeros((keys.shape[0], 8), keys.dtype)
  keys_p = jnp.concatenate([keys[:, :8], zeros8, keys[:, 8:]], axis=1)

  blk = 2048
  grid = n // blk
  out = pl.pallas_call(
      _router_tc_kernel(blk),
      grid=(grid,),
      in_specs=[
          pl.BlockSpec(memory_space=pltpu.SMEM),
          pl.BlockSpec((blk, _EW), lambda i: (i, 0)),
          pl.BlockSpec((keys.shape[0], _EW), lambda i: (0, 0)),
      ],
      out_specs=pl.BlockSpec((blk, 64), lambda i: (i, 0)),
      out_shape=jax.ShapeDtypeStruct((n, 64), jnp.float32),
  )(jnp.reshape(threshold, (1,)), emb, keys_p)
  return out


# speedup vs baseline: 8.5268x; 8.5268x over previous
import jax
import jax.numpy as jnp
from jax import lax
from jax.experimental import pallas as pl
from jax.experimental.pallas import tpu as pltpu


def _body(thr_ref, x_ref, keys_ref, out_ref):
  x = x_ref[...]
  bn = x.shape[1]
  r = x.reshape(8, 4, 4, 4, 4, bn)
  p4 = r.max(axis=4).max(axis=2)
  r2 = p4.reshape(8, 2, 2, 2, 2, bn)
  p2 = r2.max(axis=4).max(axis=2)
  p1 = p2.max(axis=2).max(axis=1)
  emb = jnp.concatenate(
      [p1, p2.reshape(32, bn), p4.reshape(128, bn)], axis=0)
  s = jnp.sum(emb * emb, axis=0, keepdims=True)
  emb = emb / jnp.maximum(jnp.sqrt(s), 1e-12)
  logits = lax.dot_general(
      keys_ref[...], emb, (((1,), (0,)), ((), ())),
      preferred_element_type=jnp.float32)
  m = jnp.max(logits, axis=0, keepdims=True)
  e = jnp.exp(logits - m)
  w = e / jnp.sum(e, axis=0, keepdims=True)
  t = thr_ref[0]
  wf = jnp.where(w > t, w, 0.0)
  out_ref[...] = wf / (jnp.sum(wf, axis=0, keepdims=True) + 1e-8)


def kernel(patch, keys, threshold):
  n = patch.shape[0]
  pt = jnp.transpose(patch, (1, 2, 3, 0)).reshape(2048, n)
  bn = 1024
  grid = n // bn
  out_t = pl.pallas_call(
      _body,
      grid=(grid,),
      in_specs=[
          pl.BlockSpec(memory_space=pltpu.SMEM),
          pl.BlockSpec((2048, bn), lambda i: (0, i)),
          pl.BlockSpec((64, 168), lambda i: (0, 0)),
      ],
      out_specs=pl.BlockSpec((64, bn), lambda i: (0, i)),
      out_shape=jax.ShapeDtypeStruct((64, n), jnp.float32),
  )(jnp.reshape(threshold, (1,)), pt, keys)
  return jnp.transpose(out_t)


# speedup vs baseline: 9.1595x; 1.0742x over previous
import functools

import jax
import jax.numpy as jnp
from jax import lax
from jax.experimental import pallas as pl
from jax.experimental.pallas import tpu as pltpu
from jax.experimental.pallas import tpu_sc as plsc

_NC = 2
_NS = 16
_NW = _NC * _NS
_BN = 128
_C = 8
_ROWS = 2048
_ED = 168


def _ssp_sc_kernel(n: int):
  blocks_per_w = n // _BN // _NW
  mesh = plsc.VectorSubcoreMesh(core_axis_name="c", subcore_axis_name="s")

  @functools.partial(
      pl.kernel,
      out_type=jax.ShapeDtypeStruct((_ED, n), jnp.float32),
      mesh=mesh,
      scratch_types=[
          pltpu.VMEM((256, _BN), jnp.float32),
          pltpu.VMEM((256, _BN), jnp.float32),
          pltpu.VMEM((_ED, _BN), jnp.float32),
          pltpu.SemaphoreType.DMA,
          pltpu.SemaphoreType.DMA,
      ],
      compiler_params=pltpu.CompilerParams(needs_layout_passes=False),
  )
  def ssp(pt_hbm, emb_hbm, buf0, buf1, ebuf, sem0, sem1):
    wid = lax.axis_index("s") * _NC + lax.axis_index("c")
    bufs = (buf0, buf1)
    sems = (sem0, sem1)

    def block_body(k, carry):
      n0 = (wid * blocks_per_w + k) * _BN

      def start(c, slot):
        return pltpu.async_copy(
            pt_hbm.at[pl.ds(c * 256, 256), pl.ds(n0, _BN)], bufs[slot],
            sems[slot])

      desc = start(0, 0)
      for c in range(_C):
        desc.wait()
        if c + 1 < _C:
          desc = start(c + 1, (c + 1) % 2)
        buf = bufs[c % 2]

        def gj_body(gj, carry2):
          g = gj // 4
          j4 = gj % 4
          accs = []
          for v in range(_BN // 16):
            acc = None
            for dy in range(4):
              for dx in range(4):
                val = buf[(g * 4 + dy) * 16 + j4 * 4 + dx, pl.ds(v * 16, 16)]
                acc = val if acc is None else jnp.maximum(acc, val)
            accs.append(acc)
          for v in range(_BN // 16):
            ebuf[40 + c * 16 + gj, pl.ds(v * 16, 16)] = accs[v]
          return carry2

        lax.fori_loop(0, 16, gj_body, 0, unroll=False)

        for i in range(2):
          for j in range(2):
            for v in range(_BN // 16):
              acc = None
              for di in range(2):
                for dj in range(2):
                  val = ebuf[40 + c * 16 + (2 * i + di) * 4 + (2 * j + dj),
                             pl.ds(v * 16, 16)]
                  acc = val if acc is None else jnp.maximum(acc, val)
              ebuf[8 + c * 4 + i * 2 + j, pl.ds(v * 16, 16)] = acc
        for v in range(_BN // 16):
          acc = None
          for q in range(4):
            val = ebuf[8 + c * 4 + q, pl.ds(v * 16, 16)]
            acc = val if acc is None else jnp.maximum(acc, val)
          ebuf[c, pl.ds(v * 16, 16)] = acc

      pltpu.sync_copy(ebuf, emb_hbm.at[:, pl.ds(n0, _BN)])
      return carry

    lax.fori_loop(0, blocks_per_w, block_body, 0, unroll=False)

  return ssp


def _router_body(thr_ref, emb_ref, keys_ref, out_ref):
  emb = emb_ref[...]
  s = jnp.sum(emb * emb, axis=0, keepdims=True)
  emb = emb / jnp.maximum(jnp.sqrt(s), 1e-12)
  logits = lax.dot_general(
      keys_ref[...], emb, (((1,), (0,)), ((), ())),
      preferred_element_type=jnp.float32)
  m = jnp.max(logits, axis=0, keepdims=True)
  e = jnp.exp(logits - m)
  w = e / jnp.sum(e, axis=0, keepdims=True)
  t = thr_ref[0]
  wf = jnp.where(w > t, w, 0.0)
  out_ref[...] = wf / (jnp.sum(wf, axis=0, keepdims=True) + 1e-8)


def kernel(patch, keys, threshold):
  n = patch.shape[0]
  pt = jnp.transpose(patch, (1, 2, 3, 0)).reshape(_ROWS, n)

  emb = _ssp_sc_kernel(n)(pt)

  bn = 2048
  out_t = pl.pallas_call(
      _router_body,
      grid=(n // bn,),
      in_specs=[
          pl.BlockSpec(memory_space=pltpu.SMEM),
          pl.BlockSpec((_ED, bn), lambda i: (0, i)),
          pl.BlockSpec((64, _ED), lambda i: (0, 0)),
      ],
      out_specs=pl.BlockSpec((64, bn), lambda i: (0, i)),
      out_shape=jax.ShapeDtypeStruct((64, n), jnp.float32),
  )(jnp.reshape(threshold, (1,)), emb, keys)
  return jnp.transpose(out_t)
